# Initial kernel scaffold; baseline (speedup 1.0000x reference)
import jax
import jax.numpy as jnp
from jax import lax
from jax.experimental import pallas as pl
from jax.experimental.pallas import tpu as pltpu
from jax.experimental.pallas import tpu_sc as plsc

N_NODES = 10000
N_EDGES = 160000
D_EDGE = 16
WIDTH = 32

NC = 2
NS = 16
NW = NC * NS

EW_G = N_EDGES // NW
C_G = 1000
NCH_G = EW_G // C_G


def _gather_body(x_hbm, snd_hbm, out_hbm, idx_v, rows_v, sem):
    wid = lax.axis_index("s") * NC + lax.axis_index("c")
    base = wid * EW_G

    def chunk(i, carry):
        off = base + i * C_G
        pltpu.sync_copy(snd_hbm.at[pl.ds(off, C_G)], idx_v)
        pltpu.async_copy(x_hbm.at[idx_v], rows_v, sem).wait()
        pltpu.sync_copy(rows_v, out_hbm.at[pl.ds(off, C_G)])
        return carry

    lax.fori_loop(0, NCH_G, chunk, 0)


def _sc_gather(x, senders):
    mesh = plsc.VectorSubcoreMesh(core_axis_name="c", subcore_axis_name="s")
    return pl.kernel(
        _gather_body,
        out_type=jax.ShapeDtypeStruct((N_EDGES, WIDTH), jnp.float32),
        mesh=mesh,
        scratch_types=[
            pltpu.VMEM((C_G,), jnp.int32),
            pltpu.VMEM((C_G, WIDTH), jnp.float32),
            pltpu.SemaphoreType.DMA,
        ],
    )(x, senders)


BE = 2000
GRID_E = N_EDGES // BE


def _mm_body(a_ref, xj_ref, w_ref, o_ref):
    a = a_ref[...]
    xj = xj_ref[...]
    parts = [a[:, d : d + 1] * xj for d in range(D_EDGE)]
    z = jnp.concatenate(parts + [xj], axis=1)
    o_ref[...] = jnp.dot(z, w_ref[...], preferred_element_type=jnp.float32)


def _tc_matmul(edge_attr, x_j, w_full):
    return pl.pallas_call(
        _mm_body,
        grid=(GRID_E,),
        in_specs=[
            pl.BlockSpec((BE, D_EDGE), lambda i: (i, 0)),
            pl.BlockSpec((BE, WIDTH), lambda i: (i, 0)),
            pl.BlockSpec((D_EDGE * WIDTH + WIDTH, WIDTH), lambda i: (0, 0)),
        ],
        out_specs=pl.BlockSpec((BE, WIDTH), lambda i: (i, 0)),
        out_shape=jax.ShapeDtypeStruct((N_EDGES, WIDTH), jnp.float32),
        compiler_params=pltpu.CompilerParams(
            dimension_semantics=("arbitrary",),
        ),
    )(edge_attr, x_j, w_full)


HALF = N_NODES // NC
ACC_ROWS = 5120
DUMP = HALF
EW_S = N_EDGES // NS
C_S = 1000
NCH_S = EW_S // C_S
CPAD = 1008
ROWS_T = 312


def _scatter_body(msg_hbm, rcv_hbm, out_hbm, idx_v, m_v, buf_v, acc_sh):
    cid = lax.axis_index("c")
    sid = lax.axis_index("s")
    lo = cid * HALF

    def z_row(i, carry):
        buf_v[i, pl.ds(0, 16)] = jnp.zeros((16,), jnp.float32)
        buf_v[i, pl.ds(16, 16)] = jnp.zeros((16,), jnp.float32)
        return carry

    lax.fori_loop(0, 320, z_row, 0)
    pltpu.sync_copy(buf_v, acc_sh.at[pl.ds(sid * 320, 320)])
    plsc.subcore_barrier()

    lane = lax.iota(jnp.int32, 16)
    base = sid * EW_S

    def chunk(ci, carry):
        off = base + ci * C_S
        pltpu.sync_copy(rcv_hbm.at[pl.ds(off, C_S)], idx_v.at[pl.ds(0, C_S)])
        pltpu.sync_copy(msg_hbm.at[pl.ds(off, C_S)], m_v.at[pl.ds(0, C_S)])

        def xform(j, c2):
            r = idx_v[pl.ds(j * 16, 16)]
            valid = (j * 16 + lane < C_S) & (r >= lo) & (r < lo + HALF)
            idx_v[pl.ds(j * 16, 16)] = jnp.where(valid, r - lo, DUMP)
            return c2

        lax.fori_loop(0, CPAD // 16, xform, 0)
        pltpu.sync_copy(m_v, acc_sh.at[idx_v], add=True)
        return carry

    lax.fori_loop(0, NCH_S, chunk, 0)
    plsc.subcore_barrier()

    @pl.when(sid < NS - 1)
    def _():
        pltpu.sync_copy(acc_sh.at[pl.ds(sid * ROWS_T, ROWS_T)],
                        buf_v.at[pl.ds(0, ROWS_T)])
        pltpu.sync_copy(buf_v.at[pl.ds(0, ROWS_T)],
                        out_hbm.at[pl.ds(lo + sid * ROWS_T, ROWS_T)])

    @pl.when(sid == NS - 1)
    def _():
        pltpu.sync_copy(acc_sh.at[pl.ds(15 * ROWS_T, 320)], buf_v)
        pltpu.sync_copy(buf_v, out_hbm.at[pl.ds(lo + 15 * ROWS_T, 320)])


def _sc_scatter(msgs, receivers):
    mesh = plsc.VectorSubcoreMesh(core_axis_name="c", subcore_axis_name="s")
    return pl.kernel(
        _scatter_body,
        out_type=jax.ShapeDtypeStruct((N_NODES, WIDTH), jnp.float32),
        mesh=mesh,
        scratch_types=[
            pltpu.VMEM((CPAD,), jnp.int32),
            pltpu.VMEM((CPAD, WIDTH), jnp.float32),
            pltpu.VMEM((320, WIDTH), jnp.float32),
            pltpu.VMEM_SHARED((ACC_ROWS, WIDTH), jnp.float32),
        ],
    )(msgs, receivers)


def kernel(x, senders, receivers, edge_attr, W_nn, b_nn):
    w_r = W_nn.reshape(D_EDGE * WIDTH, WIDTH)
    b_r = b_nn.reshape(WIDTH, WIDTH)
    w_full = jnp.concatenate([w_r, b_r], axis=0)
    x_j = _sc_gather(x, senders)
    msgs = _tc_matmul(edge_attr, x_j, w_full)
    return _sc_scatter(msgs, receivers)


# speedup vs baseline: 1.8129x; 1.8129x over previous
import jax
import jax.numpy as jnp
from jax import lax
from jax.experimental import pallas as pl
from jax.experimental.pallas import tpu as pltpu
from jax.experimental.pallas import tpu_sc as plsc

N_NODES = 10000
N_EDGES = 160000
D_EDGE = 16
WIDTH = 32

NC = 2
NS = 16
NW = NC * NS

EW_G = N_EDGES // NW
C_G = 1000
NCH_G = EW_G // C_G


def _gather_body(x_hbm, snd_hbm, out_hbm, idx_v, rows_v, sem):
    wid = lax.axis_index("s") * NC + lax.axis_index("c")
    base = wid * EW_G

    def chunk(i, carry):
        off = base + i * C_G
        pltpu.sync_copy(snd_hbm.at[pl.ds(off, C_G)], idx_v)
        pltpu.async_copy(x_hbm.at[idx_v], rows_v, sem).wait()
        pltpu.sync_copy(rows_v, out_hbm.at[pl.ds(off, C_G)])
        return carry

    lax.fori_loop(0, NCH_G, chunk, 0)


def _sc_gather(x, senders):
    mesh = plsc.VectorSubcoreMesh(core_axis_name="c", subcore_axis_name="s")
    return pl.kernel(
        _gather_body,
        out_type=jax.ShapeDtypeStruct((N_EDGES, WIDTH), jnp.float32),
        mesh=mesh,
        compiler_params=pltpu.CompilerParams(use_tc_tiling_on_sc=False),
        scratch_types=[
            pltpu.VMEM((C_G,), jnp.int32),
            pltpu.VMEM((C_G, WIDTH), jnp.float32),
            pltpu.SemaphoreType.DMA,
        ],
    )(x, senders)


BE = 2000
GRID_E = N_EDGES // BE


def _mm_body(a_ref, xj_ref, w_ref, o_ref):
    a = a_ref[...]
    xj = xj_ref[...]
    parts = [a[:, d : d + 1] * xj for d in range(D_EDGE)]
    z = jnp.concatenate(parts + [xj], axis=1)
    o_ref[...] = jnp.dot(z, w_ref[...], preferred_element_type=jnp.float32)


def _tc_matmul(edge_attr, x_j, w_full):
    return pl.pallas_call(
        _mm_body,
        grid=(GRID_E,),
        in_specs=[
            pl.BlockSpec((BE, D_EDGE), lambda i: (i, 0)),
            pl.BlockSpec((BE, WIDTH), lambda i: (i, 0)),
            pl.BlockSpec((D_EDGE * WIDTH + WIDTH, WIDTH), lambda i: (0, 0)),
        ],
        out_specs=pl.BlockSpec((BE, WIDTH), lambda i: (i, 0)),
        out_shape=jax.ShapeDtypeStruct((N_EDGES, WIDTH), jnp.float32),
        compiler_params=pltpu.CompilerParams(
            dimension_semantics=("arbitrary",),
        ),
    )(edge_attr, x_j, w_full)


HALF = N_NODES // NC
ACC_ROWS = 5120
DUMP = HALF
EW_S = N_EDGES // NS
C_S = 1000
NCH_S = EW_S // C_S
CPAD = 1008
ROWS_T = 312


def _scatter_body(msg_hbm, rcv_hbm, out_hbm, idx_v, m_v, buf_v, acc_sh):
    cid = lax.axis_index("c")
    sid = lax.axis_index("s")
    lo = cid * HALF

    def z_row(i, carry):
        buf_v[i, pl.ds(0, 16)] = jnp.zeros((16,), jnp.float32)
        buf_v[i, pl.ds(16, 16)] = jnp.zeros((16,), jnp.float32)
        return carry

    lax.fori_loop(0, 320, z_row, 0)
    pltpu.sync_copy(buf_v, acc_sh.at[pl.ds(sid * 320, 320)])
    plsc.subcore_barrier()

    lane = lax.iota(jnp.int32, 16)
    base = sid * EW_S

    def chunk(ci, carry):
        off = base + ci * C_S
        pltpu.sync_copy(rcv_hbm.at[pl.ds(off, C_S)], idx_v.at[pl.ds(0, C_S)])
        pltpu.sync_copy(msg_hbm.at[pl.ds(off, C_S)], m_v.at[pl.ds(0, C_S)])

        def xform(j, c2):
            r = idx_v[pl.ds(j * 16, 16)]
            valid = (j * 16 + lane < C_S) & (r >= lo) & (r < lo + HALF)
            idx_v[pl.ds(j * 16, 16)] = jnp.where(valid, r - lo, DUMP)
            return c2

        lax.fori_loop(0, CPAD // 16, xform, 0)
        pltpu.sync_copy(m_v, acc_sh.at[idx_v], add=True)
        return carry

    lax.fori_loop(0, NCH_S, chunk, 0)
    plsc.subcore_barrier()

    @pl.when(sid < NS - 1)
    def _():
        pltpu.sync_copy(acc_sh.at[pl.ds(sid * ROWS_T, ROWS_T)],
                        buf_v.at[pl.ds(0, ROWS_T)])
        pltpu.sync_copy(buf_v.at[pl.ds(0, ROWS_T)],
                        out_hbm.at[pl.ds(lo + sid * ROWS_T, ROWS_T)])

    @pl.when(sid == NS - 1)
    def _():
        pltpu.sync_copy(acc_sh.at[pl.ds(15 * ROWS_T, 320)], buf_v)
        pltpu.sync_copy(buf_v, out_hbm.at[pl.ds(lo + 15 * ROWS_T, 320)])


def _sc_scatter(msgs, receivers):
    mesh = plsc.VectorSubcoreMesh(core_axis_name="c", subcore_axis_name="s")
    return pl.kernel(
        _scatter_body,
        out_type=jax.ShapeDtypeStruct((N_NODES, WIDTH), jnp.float32),
        mesh=mesh,
        compiler_params=pltpu.CompilerParams(use_tc_tiling_on_sc=False),
        scratch_types=[
            pltpu.VMEM((CPAD,), jnp.int32),
            pltpu.VMEM((CPAD, WIDTH), jnp.float32),
            pltpu.VMEM((320, WIDTH), jnp.float32),
            pltpu.VMEM_SHARED((ACC_ROWS, WIDTH), jnp.float32),
        ],
    )(msgs, receivers)


def kernel(x, senders, receivers, edge_attr, W_nn, b_nn):
    w_r = W_nn.reshape(D_EDGE * WIDTH, WIDTH)
    b_r = b_nn.reshape(WIDTH, WIDTH)
    w_full = jnp.concatenate([w_r, b_r], axis=0)
    x_j = _sc_gather(x, senders)
    msgs = _tc_matmul(edge_attr, x_j, w_full)
    return _sc_scatter(msgs, receivers)


# speedup vs baseline: 3.4825x; 1.9210x over previous
import jax
import jax.numpy as jnp
from jax import lax
from jax.experimental import pallas as pl
from jax.experimental.pallas import tpu as pltpu
from jax.experimental.pallas import tpu_sc as plsc

N_NODES = 10000
N_EDGES = 160000
D_EDGE = 16
WIDTH = 32

NC = 2
NS = 16
NW = NC * NS

EW_G = N_EDGES // NW
C_G = 1000
NCH_G = EW_G // C_G


def _gather_body(x_hbm, snd_hbm, out_hbm, idx_v, rows_v, sem):
    wid = lax.axis_index("s") * NC + lax.axis_index("c")
    base = wid * EW_G

    def chunk(i, carry):
        off = base + i * C_G
        pltpu.sync_copy(snd_hbm.at[pl.ds(off, C_G)], idx_v)
        pltpu.async_copy(x_hbm.at[idx_v], rows_v, sem).wait()
        pltpu.sync_copy(rows_v, out_hbm.at[pl.ds(off, C_G)])
        return carry

    lax.fori_loop(0, NCH_G, chunk, 0)


def _sc_gather(x, senders):
    mesh = plsc.VectorSubcoreMesh(core_axis_name="c", subcore_axis_name="s")
    return pl.kernel(
        _gather_body,
        out_type=jax.ShapeDtypeStruct((N_EDGES, WIDTH), jnp.float32),
        mesh=mesh,
        compiler_params=pltpu.CompilerParams(use_tc_tiling_on_sc=False),
        scratch_types=[
            pltpu.VMEM((C_G,), jnp.int32),
            pltpu.VMEM((C_G, WIDTH), jnp.float32),
            pltpu.SemaphoreType.DMA,
        ],
    )(x, senders)


BE = 2000
GRID_E = N_EDGES // BE


def _mm_body(a_ref, xj_ref, s_ref, w_ref, o_ref):
    a = a_ref[...].astype(jnp.bfloat16)
    xj = xj_ref[...].astype(jnp.bfloat16)
    a_rep = jnp.dot(a, s_ref[...],
                    preferred_element_type=jnp.float32).astype(jnp.bfloat16)
    xt = jnp.concatenate([xj] * D_EDGE, axis=1)
    z = jnp.concatenate([a_rep * xt, xj], axis=1)
    o_ref[...] = jnp.dot(z, w_ref[...], preferred_element_type=jnp.float32)


def _tc_matmul(edge_attr, x_j, s_mat, w_full):
    return pl.pallas_call(
        _mm_body,
        grid=(GRID_E,),
        in_specs=[
            pl.BlockSpec((BE, D_EDGE), lambda i: (i, 0)),
            pl.BlockSpec((BE, WIDTH), lambda i: (i, 0)),
            pl.BlockSpec((D_EDGE, D_EDGE * WIDTH), lambda i: (0, 0)),
            pl.BlockSpec((D_EDGE * WIDTH + WIDTH, WIDTH), lambda i: (0, 0)),
        ],
        out_specs=pl.BlockSpec((BE, WIDTH), lambda i: (i, 0)),
        out_shape=jax.ShapeDtypeStruct((N_EDGES, WIDTH), jnp.float32),
        compiler_params=pltpu.CompilerParams(
            dimension_semantics=("arbitrary",),
        ),
    )(edge_attr, x_j, s_mat, w_full)


HALF = N_NODES // NC
ACC_ROWS = 5120
DUMP = HALF
EW_S = N_EDGES // NS
C_S = 1000
NCH_S = EW_S // C_S
CPAD = 1008
ROWS_T = 312


def _scatter_body(msg_hbm, rcv_hbm, out_hbm, idx_v, m_v, buf_v, acc_sh):
    cid = lax.axis_index("c")
    sid = lax.axis_index("s")
    lo = cid * HALF

    def z_row(i, carry):
        buf_v[i, pl.ds(0, 16)] = jnp.zeros((16,), jnp.float32)
        buf_v[i, pl.ds(16, 16)] = jnp.zeros((16,), jnp.float32)
        return carry

    lax.fori_loop(0, 320, z_row, 0)
    pltpu.sync_copy(buf_v, acc_sh.at[pl.ds(sid * 320, 320)])
    plsc.subcore_barrier()

    lane = lax.iota(jnp.int32, 16)
    base = sid * EW_S

    def chunk(ci, carry):
        off = base + ci * C_S
        pltpu.sync_copy(rcv_hbm.at[pl.ds(off, C_S)], idx_v.at[pl.ds(0, C_S)])
        pltpu.sync_copy(msg_hbm.at[pl.ds(off, C_S)], m_v.at[pl.ds(0, C_S)])

        def xform(j, c2):
            r = idx_v[pl.ds(j * 16, 16)]
            valid = (j * 16 + lane < C_S) & (r >= lo) & (r < lo + HALF)
            idx_v[pl.ds(j * 16, 16)] = jnp.where(valid, r - lo, DUMP)
            return c2

        lax.fori_loop(0, CPAD // 16, xform, 0)
        pltpu.sync_copy(m_v, acc_sh.at[idx_v], add=True)
        return carry

    lax.fori_loop(0, NCH_S, chunk, 0)
    plsc.subcore_barrier()

    @pl.when(sid < NS - 1)
    def _():
        pltpu.sync_copy(acc_sh.at[pl.ds(sid * ROWS_T, ROWS_T)],
                        buf_v.at[pl.ds(0, ROWS_T)])
        pltpu.sync_copy(buf_v.at[pl.ds(0, ROWS_T)],
                        out_hbm.at[pl.ds(lo + sid * ROWS_T, ROWS_T)])

    @pl.when(sid == NS - 1)
    def _():
        pltpu.sync_copy(acc_sh.at[pl.ds(15 * ROWS_T, 320)], buf_v)
        pltpu.sync_copy(buf_v, out_hbm.at[pl.ds(lo + 15 * ROWS_T, 320)])


def _sc_scatter(msgs, receivers):
    mesh = plsc.VectorSubcoreMesh(core_axis_name="c", subcore_axis_name="s")
    return pl.kernel(
        _scatter_body,
        out_type=jax.ShapeDtypeStruct((N_NODES, WIDTH), jnp.float32),
        mesh=mesh,
        compiler_params=pltpu.CompilerParams(use_tc_tiling_on_sc=False),
        scratch_types=[
            pltpu.VMEM((CPAD,), jnp.int32),
            pltpu.VMEM((CPAD, WIDTH), jnp.float32),
            pltpu.VMEM((320, WIDTH), jnp.float32),
            pltpu.VMEM_SHARED((ACC_ROWS, WIDTH), jnp.float32),
        ],
    )(msgs, receivers)


def kernel(x, senders, receivers, edge_attr, W_nn, b_nn):
    w_r = W_nn.reshape(D_EDGE * WIDTH, WIDTH)
    b_r = b_nn.reshape(WIDTH, WIDTH)
    w_full = jnp.concatenate([w_r, b_r], axis=0).astype(jnp.bfloat16)
    s_mat = (jnp.arange(D_EDGE * WIDTH)[None, :] // WIDTH
             == jnp.arange(D_EDGE)[:, None]).astype(jnp.bfloat16)
    x_j = _sc_gather(x, senders)
    msgs = _tc_matmul(edge_attr, x_j, s_mat, w_full)
    return _sc_scatter(msgs, receivers)


# speedup vs baseline: 3.7742x; 1.0838x over previous
import jax
import jax.numpy as jnp
from jax import lax
from jax.experimental import pallas as pl
from jax.experimental.pallas import tpu as pltpu
from jax.experimental.pallas import tpu_sc as plsc

N_NODES = 10000
N_EDGES = 160000
D_EDGE = 16
WIDTH = 32

NC = 2
NS = 16
NW = NC * NS

EW_G = N_EDGES // NW
C_G = 1000
NCH_G = EW_G // C_G


NSLOT = 3


def _gather_body(x_hbm, snd_hbm, out_hbm, idx_v, r0, r1, r2,
                 g0, g1, g2, w0, w1, w2):
    rows = [r0, r1, r2]
    gsem = [g0, g1, g2]
    wsem = [w0, w1, w2]
    wid = lax.axis_index("s") * NC + lax.axis_index("c")
    base = wid * EW_G

    pltpu.sync_copy(snd_hbm.at[pl.ds(base, EW_G)], idx_v)

    def start_g(i):
        s = i % NSLOT
        return pltpu.async_copy(
            x_hbm.at[idx_v.at[pl.ds(i * C_G, C_G)]], rows[s], gsem[s])

    gets = {i: start_g(i) for i in range(min(NSLOT, NCH_G))}
    puts = {}
    for i in range(NCH_G):
        s = i % NSLOT
        gets[i].wait()
        puts[i] = pltpu.async_copy(
            rows[s], out_hbm.at[pl.ds(base + i * C_G, C_G)], wsem[s])
        if i + NSLOT < NCH_G:
            puts[i].wait()
            gets[i + NSLOT] = start_g(i + NSLOT)
    for i in range(max(0, NCH_G - NSLOT), NCH_G):
        puts[i].wait()


def _sc_gather(x, senders):
    mesh = plsc.VectorSubcoreMesh(core_axis_name="c", subcore_axis_name="s")
    return pl.kernel(
        _gather_body,
        out_type=jax.ShapeDtypeStruct((N_EDGES, WIDTH), jnp.float32),
        mesh=mesh,
        compiler_params=pltpu.CompilerParams(use_tc_tiling_on_sc=False),
        scratch_types=[
            pltpu.VMEM((EW_G,), jnp.int32),
            pltpu.VMEM((C_G, WIDTH), jnp.float32),
            pltpu.VMEM((C_G, WIDTH), jnp.float32),
            pltpu.VMEM((C_G, WIDTH), jnp.float32),
            pltpu.SemaphoreType.DMA,
            pltpu.SemaphoreType.DMA,
            pltpu.SemaphoreType.DMA,
            pltpu.SemaphoreType.DMA,
            pltpu.SemaphoreType.DMA,
            pltpu.SemaphoreType.DMA,
        ],
    )(x, senders)


BE = 4000
GRID_E = N_EDGES // BE


def _mm_body(a_ref, xj_ref, s_ref, w_ref, o_ref):
    a = a_ref[...].astype(jnp.bfloat16)
    xj = xj_ref[...].astype(jnp.bfloat16)
    a_rep = jnp.dot(a, s_ref[...],
                    preferred_element_type=jnp.float32).astype(jnp.bfloat16)
    xt = jnp.concatenate([xj] * D_EDGE, axis=1)
    z = jnp.concatenate([a_rep * xt, xj], axis=1)
    o_ref[...] = jnp.dot(z, w_ref[...], preferred_element_type=jnp.float32)


def _tc_matmul(edge_attr, x_j, s_mat, w_full):
    return pl.pallas_call(
        _mm_body,
        grid=(GRID_E,),
        in_specs=[
            pl.BlockSpec((BE, D_EDGE), lambda i: (i, 0)),
            pl.BlockSpec((BE, WIDTH), lambda i: (i, 0)),
            pl.BlockSpec((D_EDGE, D_EDGE * WIDTH), lambda i: (0, 0)),
            pl.BlockSpec((D_EDGE * WIDTH + WIDTH, WIDTH), lambda i: (0, 0)),
        ],
        out_specs=pl.BlockSpec((BE, WIDTH), lambda i: (i, 0)),
        out_shape=jax.ShapeDtypeStruct((N_EDGES, WIDTH), jnp.float32),
        compiler_params=pltpu.CompilerParams(
            dimension_semantics=("arbitrary",),
        ),
    )(edge_attr, x_j, s_mat, w_full)


HALF = N_NODES // NC
ACC_ROWS = 5120
DUMP = HALF
EW_S = N_EDGES // NS
C_S = 1000
NCH_S = EW_S // C_S
CPAD = 1008
ROWS_T = 312


def _scatter_body(msg_hbm, rcv_hbm, out_hbm, idx_v, m_v, buf_v, acc_sh):
    cid = lax.axis_index("c")
    sid = lax.axis_index("s")
    lo = cid * HALF

    def z_row(i, carry):
        buf_v[i, pl.ds(0, 16)] = jnp.zeros((16,), jnp.float32)
        buf_v[i, pl.ds(16, 16)] = jnp.zeros((16,), jnp.float32)
        return carry

    lax.fori_loop(0, 320, z_row, 0)
    pltpu.sync_copy(buf_v, acc_sh.at[pl.ds(sid * 320, 320)])
    plsc.subcore_barrier()

    lane = lax.iota(jnp.int32, 16)
    base = sid * EW_S

    def chunk(ci, carry):
        off = base + ci * C_S
        pltpu.sync_copy(rcv_hbm.at[pl.ds(off, C_S)], idx_v.at[pl.ds(0, C_S)])
        pltpu.sync_copy(msg_hbm.at[pl.ds(off, C_S)], m_v.at[pl.ds(0, C_S)])

        def xform(j, c2):
            r = idx_v[pl.ds(j * 16, 16)]
            valid = (j * 16 + lane < C_S) & (r >= lo) & (r < lo + HALF)
            idx_v[pl.ds(j * 16, 16)] = jnp.where(valid, r - lo, DUMP)
            return c2

        lax.fori_loop(0, CPAD // 16, xform, 0)
        pltpu.sync_copy(m_v, acc_sh.at[idx_v], add=True)
        return carry

    lax.fori_loop(0, NCH_S, chunk, 0)
    plsc.subcore_barrier()

    @pl.when(sid < NS - 1)
    def _():
        pltpu.sync_copy(acc_sh.at[pl.ds(sid * ROWS_T, ROWS_T)],
                        buf_v.at[pl.ds(0, ROWS_T)])
        pltpu.sync_copy(buf_v.at[pl.ds(0, ROWS_T)],
                        out_hbm.at[pl.ds(lo + sid * ROWS_T, ROWS_T)])

    @pl.when(sid == NS - 1)
    def _():
        pltpu.sync_copy(acc_sh.at[pl.ds(15 * ROWS_T, 320)], buf_v)
        pltpu.sync_copy(buf_v, out_hbm.at[pl.ds(lo + 15 * ROWS_T, 320)])


def _sc_scatter(msgs, receivers):
    mesh = plsc.VectorSubcoreMesh(core_axis_name="c", subcore_axis_name="s")
    return pl.kernel(
        _scatter_body,
        out_type=jax.ShapeDtypeStruct((N_NODES, WIDTH), jnp.float32),
        mesh=mesh,
        compiler_params=pltpu.CompilerParams(use_tc_tiling_on_sc=False),
        scratch_types=[
            pltpu.VMEM((CPAD,), jnp.int32),
            pltpu.VMEM((CPAD, WIDTH), jnp.float32),
            pltpu.VMEM((320, WIDTH), jnp.float32),
            pltpu.VMEM_SHARED((ACC_ROWS, WIDTH), jnp.float32),
        ],
    )(msgs, receivers)


def kernel(x, senders, receivers, edge_attr, W_nn, b_nn):
    w_r = W_nn.reshape(D_EDGE * WIDTH, WIDTH)
    b_r = b_nn.reshape(WIDTH, WIDTH)
    w_full = jnp.concatenate([w_r, b_r], axis=0).astype(jnp.bfloat16)
    s_mat = (jnp.arange(D_EDGE * WIDTH)[None, :] // WIDTH
             == jnp.arange(D_EDGE)[:, None]).astype(jnp.bfloat16)
    x_j = _sc_gather(x, senders)
    msgs = _tc_matmul(edge_attr, x_j, s_mat, w_full)
    return _sc_scatter(msgs, receivers)


# speedup vs baseline: 3.8315x; 1.0152x over previous
import jax
import jax.numpy as jnp
from jax import lax
from jax.experimental import pallas as pl
from jax.experimental.pallas import tpu as pltpu
from jax.experimental.pallas import tpu_sc as plsc

N_NODES = 10000
N_EDGES = 160000
D_EDGE = 16
WIDTH = 32

NC = 2
NS = 16
NW = NC * NS

EW_G = N_EDGES // NW
C_G = 1000
NCH_G = EW_G // C_G


NSLOT = 3


XROWS_T = N_NODES // NS


def _gather_body(x_hbm, snd_hbm, out_hbm, idx_v, r0, r1, r2, xs_sh,
                 g0, g1, g2, w0, w1, w2):
    rows = [r0, r1, r2]
    gsem = [g0, g1, g2]
    wsem = [w0, w1, w2]
    sid = lax.axis_index("s")
    wid = sid * NC + lax.axis_index("c")
    base = wid * EW_G

    pltpu.sync_copy(x_hbm.at[pl.ds(sid * XROWS_T, XROWS_T)],
                    xs_sh.at[pl.ds(sid * XROWS_T, XROWS_T)])
    pltpu.sync_copy(snd_hbm.at[pl.ds(base, EW_G)], idx_v)
    plsc.subcore_barrier()

    def start_g(i):
        s = i % NSLOT
        return pltpu.async_copy(
            xs_sh.at[idx_v.at[pl.ds(i * C_G, C_G)]], rows[s], gsem[s])

    gets = {i: start_g(i) for i in range(min(NSLOT, NCH_G))}
    puts = {}
    for i in range(NCH_G):
        s = i % NSLOT
        gets[i].wait()
        puts[i] = pltpu.async_copy(
            rows[s], out_hbm.at[pl.ds(base + i * C_G, C_G)], wsem[s])
        if i + NSLOT < NCH_G:
            puts[i].wait()
            gets[i + NSLOT] = start_g(i + NSLOT)
    for i in range(max(0, NCH_G - NSLOT), NCH_G):
        puts[i].wait()


def _sc_gather(x, senders):
    mesh = plsc.VectorSubcoreMesh(core_axis_name="c", subcore_axis_name="s")
    return pl.kernel(
        _gather_body,
        out_type=jax.ShapeDtypeStruct((N_EDGES, WIDTH), jnp.float32),
        mesh=mesh,
        compiler_params=pltpu.CompilerParams(use_tc_tiling_on_sc=False),
        scratch_types=[
            pltpu.VMEM((EW_G,), jnp.int32),
            pltpu.VMEM((C_G, WIDTH), jnp.float32),
            pltpu.VMEM((C_G, WIDTH), jnp.float32),
            pltpu.VMEM((C_G, WIDTH), jnp.float32),
            pltpu.VMEM_SHARED((N_NODES, WIDTH), jnp.float32),
            pltpu.SemaphoreType.DMA,
            pltpu.SemaphoreType.DMA,
            pltpu.SemaphoreType.DMA,
            pltpu.SemaphoreType.DMA,
            pltpu.SemaphoreType.DMA,
            pltpu.SemaphoreType.DMA,
        ],
    )(x, senders)


BE = 4000
GRID_E = N_EDGES // BE


def _mm_body(a_ref, xj_ref, s_ref, w_ref, o_ref):
    a = a_ref[...].astype(jnp.bfloat16)
    xj = xj_ref[...].astype(jnp.bfloat16)
    a_rep = jnp.dot(a, s_ref[...],
                    preferred_element_type=jnp.float32).astype(jnp.bfloat16)
    xt = jnp.concatenate([xj] * D_EDGE, axis=1)
    z = jnp.concatenate([a_rep * xt, xj], axis=1)
    o_ref[...] = jnp.dot(z, w_ref[...], preferred_element_type=jnp.float32)


def _tc_matmul(edge_attr, x_j, s_mat, w_full):
    return pl.pallas_call(
        _mm_body,
        grid=(GRID_E,),
        in_specs=[
            pl.BlockSpec((BE, D_EDGE), lambda i: (i, 0)),
            pl.BlockSpec((BE, WIDTH), lambda i: (i, 0)),
            pl.BlockSpec((D_EDGE, D_EDGE * WIDTH), lambda i: (0, 0)),
            pl.BlockSpec((D_EDGE * WIDTH + WIDTH, WIDTH), lambda i: (0, 0)),
        ],
        out_specs=pl.BlockSpec((BE, WIDTH), lambda i: (i, 0)),
        out_shape=jax.ShapeDtypeStruct((N_EDGES, WIDTH), jnp.float32),
        compiler_params=pltpu.CompilerParams(
            dimension_semantics=("arbitrary",),
        ),
    )(edge_attr, x_j, s_mat, w_full)


HALF = N_NODES // NC
ACC_ROWS = 5120
DUMP = HALF
EW_S = N_EDGES // NS
C_S = 1000
NCH_S = EW_S // C_S
CPAD = 1008
ROWS_T = 312


def _scatter_body(msg_hbm, rcv_hbm, out_hbm, idx_v, m_v, buf_v, acc_sh):
    cid = lax.axis_index("c")
    sid = lax.axis_index("s")
    lo = cid * HALF

    def z_row(i, carry):
        buf_v[i, pl.ds(0, 16)] = jnp.zeros((16,), jnp.float32)
        buf_v[i, pl.ds(16, 16)] = jnp.zeros((16,), jnp.float32)
        return carry

    lax.fori_loop(0, 320, z_row, 0)
    pltpu.sync_copy(buf_v, acc_sh.at[pl.ds(sid * 320, 320)])
    plsc.subcore_barrier()

    lane = lax.iota(jnp.int32, 16)
    base = sid * EW_S

    def chunk(ci, carry):
        off = base + ci * C_S
        pltpu.sync_copy(rcv_hbm.at[pl.ds(off, C_S)], idx_v.at[pl.ds(0, C_S)])
        pltpu.sync_copy(msg_hbm.at[pl.ds(off, C_S)], m_v.at[pl.ds(0, C_S)])

        def xform(j, c2):
            r = idx_v[pl.ds(j * 16, 16)]
            valid = (j * 16 + lane < C_S) & (r >= lo) & (r < lo + HALF)
            idx_v[pl.ds(j * 16, 16)] = jnp.where(valid, r - lo, DUMP)
            return c2

        lax.fori_loop(0, CPAD // 16, xform, 0)
        pltpu.sync_copy(m_v, acc_sh.at[idx_v], add=True)
        return carry

    lax.fori_loop(0, NCH_S, chunk, 0)
    plsc.subcore_barrier()

    @pl.when(sid < NS - 1)
    def _():
        pltpu.sync_copy(acc_sh.at[pl.ds(sid * ROWS_T, ROWS_T)],
                        buf_v.at[pl.ds(0, ROWS_T)])
        pltpu.sync_copy(buf_v.at[pl.ds(0, ROWS_T)],
                        out_hbm.at[pl.ds(lo + sid * ROWS_T, ROWS_T)])

    @pl.when(sid == NS - 1)
    def _():
        pltpu.sync_copy(acc_sh.at[pl.ds(15 * ROWS_T, 320)], buf_v)
        pltpu.sync_copy(buf_v, out_hbm.at[pl.ds(lo + 15 * ROWS_T, 320)])


def _sc_scatter(msgs, receivers):
    mesh = plsc.VectorSubcoreMesh(core_axis_name="c", subcore_axis_name="s")
    return pl.kernel(
        _scatter_body,
        out_type=jax.ShapeDtypeStruct((N_NODES, WIDTH), jnp.float32),
        mesh=mesh,
        compiler_params=pltpu.CompilerParams(use_tc_tiling_on_sc=False),
        scratch_types=[
            pltpu.VMEM((CPAD,), jnp.int32),
            pltpu.VMEM((CPAD, WIDTH), jnp.float32),
            pltpu.VMEM((320, WIDTH), jnp.float32),
            pltpu.VMEM_SHARED((ACC_ROWS, WIDTH), jnp.float32),
        ],
    )(msgs, receivers)


def kernel(x, senders, receivers, edge_attr, W_nn, b_nn):
    w_r = W_nn.reshape(D_EDGE * WIDTH, WIDTH)
    b_r = b_nn.reshape(WIDTH, WIDTH)
    w_full = jnp.concatenate([w_r, b_r], axis=0).astype(jnp.bfloat16)
    s_mat = (jnp.arange(D_EDGE * WIDTH)[None, :] // WIDTH
             == jnp.arange(D_EDGE)[:, None]).astype(jnp.bfloat16)
    x_j = _sc_gather(x, senders)
    msgs = _tc_matmul(edge_attr, x_j, s_mat, w_full)
    return _sc_scatter(msgs, receivers)


# speedup vs baseline: 4.8888x; 1.2760x over previous
import jax
import jax.numpy as jnp
from jax import lax
from jax.experimental import pallas as pl
from jax.experimental.pallas import tpu as pltpu
from jax.experimental.pallas import tpu_sc as plsc

N_NODES = 10000
N_EDGES = 160000
D_EDGE = 16
WIDTH = 32

NC = 2
NS = 16
NW = NC * NS

EW_G = N_EDGES // NW
C_G = 1000
NCH_G = EW_G // C_G


NSLOT = 3


XROWS_T = N_NODES // NS


def _gather_body(x_hbm, snd_hbm, out_hbm, idx_v, r0, r1, r2, xs_sh,
                 g0, g1, g2, w0, w1, w2):
    rows = [r0, r1, r2]
    gsem = [g0, g1, g2]
    wsem = [w0, w1, w2]
    sid = lax.axis_index("s")
    wid = sid * NC + lax.axis_index("c")
    base = wid * EW_G

    pltpu.sync_copy(x_hbm.at[pl.ds(sid * XROWS_T, XROWS_T)],
                    xs_sh.at[pl.ds(sid * XROWS_T, XROWS_T)])
    pltpu.sync_copy(snd_hbm.at[pl.ds(base, EW_G)], idx_v)
    plsc.subcore_barrier()

    def start_g(i):
        s = i % NSLOT
        return pltpu.async_copy(
            xs_sh.at[idx_v.at[pl.ds(i * C_G, C_G)]], rows[s], gsem[s])

    gets = {i: start_g(i) for i in range(min(NSLOT, NCH_G))}
    puts = {}
    for i in range(NCH_G):
        s = i % NSLOT
        gets[i].wait()
        puts[i] = pltpu.async_copy(
            rows[s], out_hbm.at[pl.ds(base + i * C_G, C_G)], wsem[s])
        if i + NSLOT < NCH_G:
            puts[i].wait()
            gets[i + NSLOT] = start_g(i + NSLOT)
    for i in range(max(0, NCH_G - NSLOT), NCH_G):
        puts[i].wait()


def _sc_gather(x, senders):
    mesh = plsc.VectorSubcoreMesh(core_axis_name="c", subcore_axis_name="s")
    return pl.kernel(
        _gather_body,
        out_type=jax.ShapeDtypeStruct((N_EDGES, WIDTH), jnp.float32),
        mesh=mesh,
        compiler_params=pltpu.CompilerParams(use_tc_tiling_on_sc=False),
        scratch_types=[
            pltpu.VMEM((EW_G,), jnp.int32),
            pltpu.VMEM((C_G, WIDTH), jnp.float32),
            pltpu.VMEM((C_G, WIDTH), jnp.float32),
            pltpu.VMEM((C_G, WIDTH), jnp.float32),
            pltpu.VMEM_SHARED((N_NODES, WIDTH), jnp.float32),
            pltpu.SemaphoreType.DMA,
            pltpu.SemaphoreType.DMA,
            pltpu.SemaphoreType.DMA,
            pltpu.SemaphoreType.DMA,
            pltpu.SemaphoreType.DMA,
            pltpu.SemaphoreType.DMA,
        ],
    )(x, senders)


PACK = 4
PB = 1000
GRID_E = N_EDGES // (PACK * PB)
KZ = D_EDGE * WIDTH + WIDTH


def _mm_body(a_ref, xp_ref, s_ref, w_ref, o_ref):
    ap = a_ref[...].astype(jnp.bfloat16)
    xp = xp_ref[...].astype(jnp.bfloat16)
    a_rep = jnp.dot(ap, s_ref[...],
                    preferred_element_type=jnp.float32).astype(jnp.bfloat16)
    pieces = []
    for c in range(PACK):
        xj = xp[:, c * WIDTH:(c + 1) * WIDTH]
        xt = jnp.concatenate([xj] * D_EDGE, axis=1)
        pieces.append(a_rep[:, c * 512:(c + 1) * 512] * xt)
        pieces.append(xj)
    z = jnp.concatenate(pieces, axis=1)
    o_ref[...] = jnp.dot(z, w_ref[...], preferred_element_type=jnp.float32)


def _tc_matmul(edge_attr_p, x_jp, s_big, w_blk):
    return pl.pallas_call(
        _mm_body,
        grid=(GRID_E,),
        in_specs=[
            pl.BlockSpec((PB, PACK * D_EDGE), lambda i: (i, 0)),
            pl.BlockSpec((PB, PACK * WIDTH), lambda i: (i, 0)),
            pl.BlockSpec((PACK * D_EDGE, PACK * 512), lambda i: (0, 0)),
            pl.BlockSpec((PACK * KZ, PACK * WIDTH), lambda i: (0, 0)),
        ],
        out_specs=pl.BlockSpec((PB, PACK * WIDTH), lambda i: (i, 0)),
        out_shape=jax.ShapeDtypeStruct((N_EDGES // PACK, PACK * WIDTH),
                                       jnp.float32),
        compiler_params=pltpu.CompilerParams(
            dimension_semantics=("arbitrary",),
        ),
    )(edge_attr_p, x_jp, s_big, w_blk)


HALF = N_NODES // NC
ACC_ROWS = 5120
DUMP = HALF
EW_S = N_EDGES // NS
C_S = 1000
NCH_S = EW_S // C_S
CPAD = 1008
ROWS_T = 312


def _scatter_body(msg_hbm, rcv_hbm, out_hbm, idx_v, m_v, buf_v, acc_sh):
    cid = lax.axis_index("c")
    sid = lax.axis_index("s")
    lo = cid * HALF

    def z_row(i, carry):
        buf_v[i, pl.ds(0, 16)] = jnp.zeros((16,), jnp.float32)
        buf_v[i, pl.ds(16, 16)] = jnp.zeros((16,), jnp.float32)
        return carry

    lax.fori_loop(0, 320, z_row, 0)
    pltpu.sync_copy(buf_v, acc_sh.at[pl.ds(sid * 320, 320)])
    plsc.subcore_barrier()

    lane = lax.iota(jnp.int32, 16)
    base = sid * EW_S

    def chunk(ci, carry):
        off = base + ci * C_S
        pltpu.sync_copy(rcv_hbm.at[pl.ds(off, C_S)], idx_v.at[pl.ds(0, C_S)])
        pltpu.sync_copy(msg_hbm.at[pl.ds(off, C_S)], m_v.at[pl.ds(0, C_S)])

        def xform(j, c2):
            r = idx_v[pl.ds(j * 16, 16)]
            valid = (j * 16 + lane < C_S) & (r >= lo) & (r < lo + HALF)
            idx_v[pl.ds(j * 16, 16)] = jnp.where(valid, r - lo, DUMP)
            return c2

        lax.fori_loop(0, CPAD // 16, xform, 0)
        pltpu.sync_copy(m_v, acc_sh.at[idx_v], add=True)
        return carry

    lax.fori_loop(0, NCH_S, chunk, 0)
    plsc.subcore_barrier()

    @pl.when(sid < NS - 1)
    def _():
        pltpu.sync_copy(acc_sh.at[pl.ds(sid * ROWS_T, ROWS_T)],
                        buf_v.at[pl.ds(0, ROWS_T)])
        pltpu.sync_copy(buf_v.at[pl.ds(0, ROWS_T)],
                        out_hbm.at[pl.ds(lo + sid * ROWS_T, ROWS_T)])

    @pl.when(sid == NS - 1)
    def _():
        pltpu.sync_copy(acc_sh.at[pl.ds(15 * ROWS_T, 320)], buf_v)
        pltpu.sync_copy(buf_v, out_hbm.at[pl.ds(lo + 15 * ROWS_T, 320)])


def _sc_scatter(msgs, receivers):
    mesh = plsc.VectorSubcoreMesh(core_axis_name="c", subcore_axis_name="s")
    return pl.kernel(
        _scatter_body,
        out_type=jax.ShapeDtypeStruct((N_NODES, WIDTH), jnp.float32),
        mesh=mesh,
        compiler_params=pltpu.CompilerParams(use_tc_tiling_on_sc=False),
        scratch_types=[
            pltpu.VMEM((CPAD,), jnp.int32),
            pltpu.VMEM((CPAD, WIDTH), jnp.float32),
            pltpu.VMEM((320, WIDTH), jnp.float32),
            pltpu.VMEM_SHARED((ACC_ROWS, WIDTH), jnp.float32),
        ],
    )(msgs, receivers)


def kernel(x, senders, receivers, edge_attr, W_nn, b_nn):
    w_r = W_nn.reshape(D_EDGE * WIDTH, WIDTH)
    b_r = b_nn.reshape(WIDTH, WIDTH)
    w_full = jnp.concatenate([w_r, b_r], axis=0).astype(jnp.bfloat16)
    w_blk = jax.scipy.linalg.block_diag(*([w_full] * PACK))
    s_mat = (jnp.arange(D_EDGE * WIDTH)[None, :] // WIDTH
             == jnp.arange(D_EDGE)[:, None]).astype(jnp.bfloat16)
    s_big = jax.scipy.linalg.block_diag(*([s_mat] * PACK))
    x_j = _sc_gather(x, senders)
    x_jp = x_j.reshape(N_EDGES // PACK, PACK * WIDTH)
    a_p = edge_attr.reshape(N_EDGES // PACK, PACK * D_EDGE)
    msgs_p = _tc_matmul(a_p, x_jp, s_big, w_blk)
    msgs = msgs_p.reshape(N_EDGES, WIDTH)
    return _sc_scatter(msgs, receivers)


# speedup vs baseline: 6.4451x; 1.3183x over previous
import jax
import jax.numpy as jnp
from jax import lax
from jax.experimental import pallas as pl
from jax.experimental.pallas import tpu as pltpu
from jax.experimental.pallas import tpu_sc as plsc

N_NODES = 10000
N_EDGES = 160000
D_EDGE = 16
WIDTH = 32

NC = 2
NS = 16
NW = NC * NS

EW_G = N_EDGES // NW
C_G = 1000
NCH_G = EW_G // C_G


NSLOT = 3


XROWS_T = N_NODES // NS


def _gather_body(x_hbm, snd_hbm, out_hbm, idx_v, r0, r1, r2, xs_sh,
                 g0, g1, g2, w0, w1, w2):
    rows = [r0, r1, r2]
    gsem = [g0, g1, g2]
    wsem = [w0, w1, w2]
    sid = lax.axis_index("s")
    wid = sid * NC + lax.axis_index("c")
    base = wid * EW_G

    pltpu.sync_copy(x_hbm.at[pl.ds(sid * XROWS_T, XROWS_T)],
                    xs_sh.at[pl.ds(sid * XROWS_T, XROWS_T)])
    pltpu.sync_copy(snd_hbm.at[pl.ds(base, EW_G)], idx_v)
    plsc.subcore_barrier()

    def start_g(i):
        s = i % NSLOT
        return pltpu.async_copy(
            xs_sh.at[idx_v.at[pl.ds(i * C_G, C_G)]], rows[s], gsem[s])

    gets = {i: start_g(i) for i in range(min(NSLOT, NCH_G))}
    puts = {}
    for i in range(NCH_G):
        s = i % NSLOT
        gets[i].wait()
        puts[i] = pltpu.async_copy(
            rows[s], out_hbm.at[pl.ds(base + i * C_G, C_G)], wsem[s])
        if i + NSLOT < NCH_G:
            puts[i].wait()
            gets[i + NSLOT] = start_g(i + NSLOT)
    for i in range(max(0, NCH_G - NSLOT), NCH_G):
        puts[i].wait()


def _sc_gather(x, senders):
    mesh = plsc.VectorSubcoreMesh(core_axis_name="c", subcore_axis_name="s")
    return pl.kernel(
        _gather_body,
        out_type=jax.ShapeDtypeStruct((N_EDGES, WIDTH), jnp.float32),
        mesh=mesh,
        compiler_params=pltpu.CompilerParams(use_tc_tiling_on_sc=False),
        scratch_types=[
            pltpu.VMEM((EW_G,), jnp.int32),
            pltpu.VMEM((C_G, WIDTH), jnp.float32),
            pltpu.VMEM((C_G, WIDTH), jnp.float32),
            pltpu.VMEM((C_G, WIDTH), jnp.float32),
            pltpu.VMEM_SHARED((N_NODES, WIDTH), jnp.float32),
            pltpu.SemaphoreType.DMA,
            pltpu.SemaphoreType.DMA,
            pltpu.SemaphoreType.DMA,
            pltpu.SemaphoreType.DMA,
            pltpu.SemaphoreType.DMA,
            pltpu.SemaphoreType.DMA,
        ],
    )(x, senders)


PACK = 4
PB = 1000
GRID_E = N_EDGES // (PACK * PB)
KZ = D_EDGE * WIDTH + WIDTH


def _mm_body(a_ref, xp_ref, s_ref, w_ref, o_ref):
    ap = a_ref[...].astype(jnp.bfloat16)
    xp = xp_ref[...].astype(jnp.bfloat16)
    a_rep = jnp.dot(ap, s_ref[...],
                    preferred_element_type=jnp.float32).astype(jnp.bfloat16)
    pieces = []
    for c in range(PACK):
        xj = xp[:, c * WIDTH:(c + 1) * WIDTH]
        xt = jnp.concatenate([xj] * D_EDGE, axis=1)
        pieces.append(a_rep[:, c * 512:(c + 1) * 512] * xt)
        pieces.append(xj)
    z = jnp.concatenate(pieces, axis=1)
    o_ref[...] = jnp.dot(z, w_ref[...], preferred_element_type=jnp.float32)


def _tc_matmul(edge_attr_p, x_jp, s_big, w_blk):
    return pl.pallas_call(
        _mm_body,
        grid=(GRID_E,),
        in_specs=[
            pl.BlockSpec((PB, PACK * D_EDGE), lambda i: (i, 0)),
            pl.BlockSpec((PB, PACK * WIDTH), lambda i: (i, 0)),
            pl.BlockSpec((PACK * D_EDGE, PACK * 512), lambda i: (0, 0)),
            pl.BlockSpec((PACK * KZ, PACK * WIDTH), lambda i: (0, 0)),
        ],
        out_specs=pl.BlockSpec((PB, PACK * WIDTH), lambda i: (i, 0)),
        out_shape=jax.ShapeDtypeStruct((N_EDGES // PACK, PACK * WIDTH),
                                       jnp.float32),
        compiler_params=pltpu.CompilerParams(
            dimension_semantics=("arbitrary",),
        ),
    )(edge_attr_p, x_jp, s_big, w_blk)


ACC_ROWS = 10240
DUMP = 10016
ZROWS = ACC_ROWS // NS
EW_S = N_EDGES // NW
C_S = 1000
NCH_S = EW_S // C_S
CPAD = 1008
ROWS_T = N_NODES // NS


def _scatter_body(msg_hbm, rcv_hbm, out_hbm, idx_v, m_v, buf_v, acc_sh):
    cid = lax.axis_index("c")
    sid = lax.axis_index("s")

    def z_row(i, carry):
        buf_v[i, pl.ds(0, 16)] = jnp.zeros((16,), jnp.float32)
        buf_v[i, pl.ds(16, 16)] = jnp.zeros((16,), jnp.float32)
        return carry

    lax.fori_loop(0, ZROWS, z_row, 0)
    pltpu.sync_copy(buf_v, acc_sh.at[pl.ds(sid * ZROWS, ZROWS)])
    plsc.subcore_barrier()

    lane = lax.iota(jnp.int32, 16)
    base = cid * (N_EDGES // NC) + sid * EW_S

    def chunk(ci, carry):
        off = base + ci * C_S
        pltpu.sync_copy(rcv_hbm.at[pl.ds(off, C_S)], idx_v.at[pl.ds(0, C_S)])
        pltpu.sync_copy(msg_hbm.at[pl.ds(off, C_S)], m_v.at[pl.ds(0, C_S)])
        tail = idx_v[pl.ds(CPAD - 16, 16)]
        idx_v[pl.ds(CPAD - 16, 16)] = jnp.where(lane < C_S - (CPAD - 16),
                                                tail, DUMP)
        pltpu.sync_copy(m_v, acc_sh.at[idx_v], add=True)
        return carry

    lax.fori_loop(0, NCH_S, chunk, 0)
    plsc.subcore_barrier()

    pltpu.sync_copy(acc_sh.at[pl.ds(sid * ROWS_T, ROWS_T)],
                    buf_v.at[pl.ds(0, ROWS_T)])
    pltpu.sync_copy(buf_v.at[pl.ds(0, ROWS_T)],
                    out_hbm.at[pl.ds(cid * N_NODES + sid * ROWS_T, ROWS_T)])


def _sc_scatter(msgs, receivers):
    mesh = plsc.VectorSubcoreMesh(core_axis_name="c", subcore_axis_name="s")
    return pl.kernel(
        _scatter_body,
        out_type=jax.ShapeDtypeStruct((NC * N_NODES, WIDTH), jnp.float32),
        mesh=mesh,
        compiler_params=pltpu.CompilerParams(use_tc_tiling_on_sc=False),
        scratch_types=[
            pltpu.VMEM((CPAD,), jnp.int32),
            pltpu.VMEM((CPAD, WIDTH), jnp.float32),
            pltpu.VMEM((ZROWS, WIDTH), jnp.float32),
            pltpu.VMEM_SHARED((ACC_ROWS, WIDTH), jnp.float32),
        ],
    )(msgs, receivers)


PN = N_NODES * WIDTH // 128


def _comb_body(p_ref, o_ref):
    o_ref[...] = p_ref[pl.ds(0, PN), :] + p_ref[pl.ds(PN, PN), :]


def _tc_combine(partials_p):
    return pl.pallas_call(
        _comb_body,
        grid=(1,),
        in_specs=[pl.BlockSpec((NC * PN, 128), lambda i: (0, 0))],
        out_specs=pl.BlockSpec((PN, 128), lambda i: (0, 0)),
        out_shape=jax.ShapeDtypeStruct((PN, 128), jnp.float32),
    )(partials_p)


def kernel(x, senders, receivers, edge_attr, W_nn, b_nn):
    w_r = W_nn.reshape(D_EDGE * WIDTH, WIDTH)
    b_r = b_nn.reshape(WIDTH, WIDTH)
    w_full = jnp.concatenate([w_r, b_r], axis=0).astype(jnp.bfloat16)
    w_blk = jax.scipy.linalg.block_diag(*([w_full] * PACK))
    s_mat = (jnp.arange(D_EDGE * WIDTH)[None, :] // WIDTH
             == jnp.arange(D_EDGE)[:, None]).astype(jnp.bfloat16)
    s_big = jax.scipy.linalg.block_diag(*([s_mat] * PACK))
    x_j = _sc_gather(x, senders)
    x_jp = x_j.reshape(N_EDGES // PACK, PACK * WIDTH)
    a_p = edge_attr.reshape(N_EDGES // PACK, PACK * D_EDGE)
    msgs_p = _tc_matmul(a_p, x_jp, s_big, w_blk)
    msgs = msgs_p.reshape(N_EDGES, WIDTH)
    partials = _sc_scatter(msgs, receivers)
    partials_p = partials.reshape(NC * PN, 128)
    return _tc_combine(partials_p).reshape(N_NODES, WIDTH)


# speedup vs baseline: 6.4951x; 1.0078x over previous
import jax
import jax.numpy as jnp
from jax import lax
from jax.experimental import pallas as pl
from jax.experimental.pallas import tpu as pltpu
from jax.experimental.pallas import tpu_sc as plsc

N_NODES = 10000
N_EDGES = 160000
D_EDGE = 16
WIDTH = 32

NC = 2
NS = 16
NW = NC * NS

EW_G = N_EDGES // NW
C_G = 1000
NCH_G = EW_G // C_G


NSLOT = 3


XROWS_T = N_NODES // NS


def _gather_body(x_hbm, snd_hbm, out_hbm, idx_v, r0, r1, r2, xs_sh,
                 g0, g1, g2, w0, w1, w2):
    rows = [r0, r1, r2]
    gsem = [g0, g1, g2]
    wsem = [w0, w1, w2]
    sid = lax.axis_index("s")
    wid = sid * NC + lax.axis_index("c")
    base = wid * EW_G

    pltpu.sync_copy(x_hbm.at[pl.ds(sid * XROWS_T, XROWS_T)],
                    xs_sh.at[pl.ds(sid * XROWS_T, XROWS_T)])
    pltpu.sync_copy(snd_hbm.at[pl.ds(base, EW_G)], idx_v)
    plsc.subcore_barrier()

    def start_g(i):
        s = i % NSLOT
        return pltpu.async_copy(
            xs_sh.at[idx_v.at[pl.ds(i * C_G, C_G)]], rows[s], gsem[s])

    gets = {i: start_g(i) for i in range(min(NSLOT, NCH_G))}
    puts = {}
    for i in range(NCH_G):
        s = i % NSLOT
        gets[i].wait()
        puts[i] = pltpu.async_copy(
            rows[s], out_hbm.at[pl.ds(base + i * C_G, C_G)], wsem[s])
        if i + NSLOT < NCH_G:
            puts[i].wait()
            gets[i + NSLOT] = start_g(i + NSLOT)
    for i in range(max(0, NCH_G - NSLOT), NCH_G):
        puts[i].wait()


def _sc_gather(x, senders):
    mesh = plsc.VectorSubcoreMesh(core_axis_name="c", subcore_axis_name="s")
    return pl.kernel(
        _gather_body,
        out_type=jax.ShapeDtypeStruct((N_EDGES, WIDTH), jnp.float32),
        mesh=mesh,
        compiler_params=pltpu.CompilerParams(use_tc_tiling_on_sc=False),
        scratch_types=[
            pltpu.VMEM((EW_G,), jnp.int32),
            pltpu.VMEM((C_G, WIDTH), jnp.float32),
            pltpu.VMEM((C_G, WIDTH), jnp.float32),
            pltpu.VMEM((C_G, WIDTH), jnp.float32),
            pltpu.VMEM_SHARED((N_NODES, WIDTH), jnp.float32),
            pltpu.SemaphoreType.DMA,
            pltpu.SemaphoreType.DMA,
            pltpu.SemaphoreType.DMA,
            pltpu.SemaphoreType.DMA,
            pltpu.SemaphoreType.DMA,
            pltpu.SemaphoreType.DMA,
        ],
    )(x, senders)


PACK = 4
PB = 1000
GRID_E = N_EDGES // (PACK * PB)
KZ = D_EDGE * WIDTH + WIDTH


def _mm_body(a_ref, xp_ref, s_ref, w_ref, o_ref):
    ap = a_ref[...].astype(jnp.bfloat16)
    xp = xp_ref[...].astype(jnp.bfloat16)
    a_rep = jnp.dot(ap, s_ref[...],
                    preferred_element_type=jnp.float32).astype(jnp.bfloat16)
    xt = jnp.concatenate(
        [jnp.concatenate([xp[:, c * WIDTH:(c + 1) * WIDTH]] * D_EDGE, axis=1)
         for c in range(PACK)], axis=1)
    z = jnp.concatenate([a_rep * xt, xp], axis=1)
    o_ref[...] = jnp.dot(z, w_ref[...], preferred_element_type=jnp.float32)


def _tc_matmul(edge_attr_p, x_jp, s_big, w_blk):
    return pl.pallas_call(
        _mm_body,
        grid=(GRID_E,),
        in_specs=[
            pl.BlockSpec((PB, PACK * D_EDGE), lambda i: (i, 0)),
            pl.BlockSpec((PB, PACK * WIDTH), lambda i: (i, 0)),
            pl.BlockSpec((PACK * D_EDGE, PACK * 512), lambda i: (0, 0)),
            pl.BlockSpec((PACK * KZ, PACK * WIDTH), lambda i: (0, 0)),
        ],
        out_specs=pl.BlockSpec((PB, PACK * WIDTH), lambda i: (i, 0)),
        out_shape=jax.ShapeDtypeStruct((N_EDGES // PACK, PACK * WIDTH),
                                       jnp.float32),
        compiler_params=pltpu.CompilerParams(
            dimension_semantics=("arbitrary",),
        ),
    )(edge_attr_p, x_jp, s_big, w_blk)


ACC_ROWS = 10240
DUMP = 10016
ZROWS = ACC_ROWS // NS
EW_S = N_EDGES // NW
C_S = 1000
NCH_S = EW_S // C_S
CPAD = 1008
ROWS_T = N_NODES // NS


def _scatter_body(msg_hbm, rcv_hbm, out_hbm, idx_v, m_v, buf_v, acc_sh):
    cid = lax.axis_index("c")
    sid = lax.axis_index("s")

    def z_row(i, carry):
        buf_v[i, pl.ds(0, 16)] = jnp.zeros((16,), jnp.float32)
        buf_v[i, pl.ds(16, 16)] = jnp.zeros((16,), jnp.float32)
        return carry

    lax.fori_loop(0, ZROWS, z_row, 0)
    pltpu.sync_copy(buf_v, acc_sh.at[pl.ds(sid * ZROWS, ZROWS)])
    plsc.subcore_barrier()

    lane = lax.iota(jnp.int32, 16)
    base = cid * (N_EDGES // NC) + sid * EW_S

    def chunk(ci, carry):
        off = base + ci * C_S
        pltpu.sync_copy(rcv_hbm.at[pl.ds(off, C_S)], idx_v.at[pl.ds(0, C_S)])
        pltpu.sync_copy(msg_hbm.at[pl.ds(off, C_S)], m_v.at[pl.ds(0, C_S)])
        tail = idx_v[pl.ds(CPAD - 16, 16)]
        idx_v[pl.ds(CPAD - 16, 16)] = jnp.where(lane < C_S - (CPAD - 16),
                                                tail, DUMP)
        pltpu.sync_copy(m_v, acc_sh.at[idx_v], add=True)
        return carry

    lax.fori_loop(0, NCH_S, chunk, 0)
    plsc.subcore_barrier()

    pltpu.sync_copy(acc_sh.at[pl.ds(sid * ROWS_T, ROWS_T)],
                    buf_v.at[pl.ds(0, ROWS_T)])
    pltpu.sync_copy(buf_v.at[pl.ds(0, ROWS_T)],
                    out_hbm.at[pl.ds(cid * N_NODES + sid * ROWS_T, ROWS_T)])


def _sc_scatter(msgs, receivers):
    mesh = plsc.VectorSubcoreMesh(core_axis_name="c", subcore_axis_name="s")
    return pl.kernel(
        _scatter_body,
        out_type=jax.ShapeDtypeStruct((NC * N_NODES, WIDTH), jnp.float32),
        mesh=mesh,
        compiler_params=pltpu.CompilerParams(use_tc_tiling_on_sc=False),
        scratch_types=[
            pltpu.VMEM((CPAD,), jnp.int32),
            pltpu.VMEM((CPAD, WIDTH), jnp.float32),
            pltpu.VMEM((ZROWS, WIDTH), jnp.float32),
            pltpu.VMEM_SHARED((ACC_ROWS, WIDTH), jnp.float32),
        ],
    )(msgs, receivers)


PN = N_NODES * WIDTH // 128


def _comb_body(p_ref, o_ref):
    o_ref[...] = p_ref[pl.ds(0, PN), :] + p_ref[pl.ds(PN, PN), :]


def _tc_combine(partials_p):
    return pl.pallas_call(
        _comb_body,
        grid=(1,),
        in_specs=[pl.BlockSpec((NC * PN, 128), lambda i: (0, 0))],
        out_specs=pl.BlockSpec((PN, 128), lambda i: (0, 0)),
        out_shape=jax.ShapeDtypeStruct((PN, 128), jnp.float32),
    )(partials_p)


def kernel(x, senders, receivers, edge_attr, W_nn, b_nn):
    w_r = W_nn.reshape(D_EDGE * WIDTH, WIDTH)
    b_r = b_nn.reshape(WIDTH, WIDTH)
    w_blk = jnp.concatenate(
        [jax.scipy.linalg.block_diag(*([w_r.astype(jnp.bfloat16)] * PACK)),
         jax.scipy.linalg.block_diag(*([b_r.astype(jnp.bfloat16)] * PACK))],
        axis=0)
    s_mat = (jnp.arange(D_EDGE * WIDTH)[None, :] // WIDTH
             == jnp.arange(D_EDGE)[:, None]).astype(jnp.bfloat16)
    s_big = jax.scipy.linalg.block_diag(*([s_mat] * PACK))
    x_j = _sc_gather(x, senders)
    x_jp = x_j.reshape(N_EDGES // PACK, PACK * WIDTH)
    a_p = edge_attr.reshape(N_EDGES // PACK, PACK * D_EDGE)
    msgs_p = _tc_matmul(a_p, x_jp, s_big, w_blk)
    msgs = msgs_p.reshape(N_EDGES, WIDTH)
    partials = _sc_scatter(msgs, receivers)
    partials_p = partials.reshape(NC * PN, 128)
    return _tc_combine(partials_p).reshape(N_NODES, WIDTH)
